# baseline (speedup 1.0000x reference)
import jax
import jax.numpy as jnp
from jax import lax
from jax.experimental import pallas as pl
from jax.experimental.pallas import tpu as pltpu
from jax.experimental.pallas import tpu_sc as plsc

B = 4096
L = 200
D = 64
EPS = 1e-5

NC = 2
NS = 16
NW = NC * NS

BW = B // NW
LPI = 2
NIT = L // LPI
ROWS = LPI * BW
TPI = LPI * (D // 8)
NT = L * (D // 8)

MAGIC = 0x5F3759DF


def _rsqrt(v):
    i = plsc.bitcast(v, jnp.int32)
    i = MAGIC - lax.shift_right_logical(i, 1)
    y = plsc.bitcast(i, jnp.float32)
    hv = 0.5 * v
    y = y * (1.5 - hv * y * y)
    y = y * (1.5 - hv * y * y)
    y = y * (1.5 - hv * y * y)
    return y


def _sc_kernel(idsT_hbm, table_hbm, pos_hbm, gam_hbm, bet_hbm, out_hbm,
               idxA, idxB, inA, inB, outA, outB, pos_v, gam_v, bet_v,
               gsemA, gsemB, osemA, osemB, isemA, isemB):
    cid = lax.axis_index("c")
    sid = lax.axis_index("s")
    wid = cid * NS + sid
    b0 = wid * BW

    idx = [idxA, idxB]
    inb = [inA, inB]
    outb = [outA, outB]
    gsem = [gsemA, gsemB]
    osem = [osemA, osemB]
    isem = [isemA, isemB]

    pltpu.sync_copy(pos_hbm.at[pl.ds(0, L), :], pos_v)
    pltpu.sync_copy(gam_hbm, gam_v)
    pltpu.sync_copy(bet_hbm, bet_v)

    g = [gam_v[pl.ds(16 * k, 16)] for k in range(4)]
    bt = [bet_v[pl.ds(16 * k, 16)] for k in range(4)]

    iota = lax.iota(jnp.int32, 16)
    rowv = [[jnp.full((16,), j * 8 + 2 * k, jnp.int32)
             + lax.shift_right_logical(iota, 3)
             for k in range(4)] for j in range(LPI)]
    colbase = jnp.bitwise_and(iota, 7) * 128

    def fire_ids(t, q):
        pltpu.async_copy(idsT_hbm.at[pl.ds(t * LPI, LPI), pl.ds(b0, BW)],
                         idx[q], isem[q])

    def wait_ids(q):
        pltpu.make_async_copy(idsT_hbm.at[pl.ds(0, LPI), pl.ds(0, BW)],
                              idx[q], isem[q]).wait()

    def fire_gather(q):
        for j in range(LPI):
            pltpu.async_copy(table_hbm.at[idx[q].at[j]],
                             inb[q].at[pl.ds(j * BW, BW), :], gsem[q])

    def wait_gather(q):
        pltpu.make_async_copy(table_hbm.at[pl.ds(0, ROWS), :], inb[q],
                              gsem[q]).wait()

    def fire_out(t, q):
        pltpu.async_copy(outb[q], out_hbm.at[pl.ds(t * TPI, TPI), wid],
                         osem[q])

    def wait_out(q):
        pltpu.make_async_copy(out_hbm.at[pl.ds(0, TPI), 0], outb[q],
                              osem[q]).wait()

    fire_ids(0, 0)
    wait_ids(0)
    fire_ids(1, 1)
    fire_gather(0)

    @pl.loop(0, NIT, step=2)
    def _iter2(t0):
        for p in (0, 1):
            t = t0 + p
            q = 1 - p

            @pl.when(t < NIT - 1)
            def _prefetch():
                wait_ids(q)
                fire_gather(q)

            wait_gather(p)

            @pl.when(t < NIT - 2)
            def _nextids():
                fire_ids(t + 2, p)

            @pl.when(t >= 2)
            def _drainout():
                wait_out(p)

            src = inb[p]
            dst = outb[p]

            for j in range(LPI):
                l = t * LPI + j
                pv = [pos_v[l, pl.ds(16 * k, 16)] for k in range(4)]
                rv = rowv[j]

                @plsc.parallel_loop(0, BW, unroll=4)
                def _row(r):
                    row = j * BW + r
                    x = [src[row, pl.ds(16 * k, 16)] + pv[k]
                         for k in range(4)]
                    tot = (x[0] + x[1]) + (x[2] + x[3])
                    qq = ((x[0] * x[0] + x[1] * x[1])
                          + (x[2] * x[2] + x[3] * x[3]))
                    sv = jnp.full((16,), jnp.sum(tot))
                    qv = jnp.full((16,), jnp.sum(qq))
                    mean = sv * (1.0 / D)
                    var = qv * (1.0 / D) - mean * mean
                    rstd = _rsqrt(var + EPS)
                    colv = colbase + r
                    for k in range(4):
                        y = (x[k] - mean) * (rstd * g[k]) + bt[k]
                        plsc.store_scatter(dst, [rv[k], colv], y)

            fire_out(t, p)

    wait_out(0)
    wait_out(1)


@jax.jit
def kernel(input_ids_BL, gene_table, pos_table, ln_gamma, ln_beta):
    ids_t = input_ids_BL.astype(jnp.int32).T

    mesh = plsc.VectorSubcoreMesh(core_axis_name="c", subcore_axis_name="s",
                                  num_cores=NC, num_subcores=NS)
    out5 = pl.kernel(
        _sc_kernel,
        out_type=jax.ShapeDtypeStruct((NT, NW, 1024), jnp.float32),
        mesh=mesh,
        compiler_params=pltpu.CompilerParams(needs_layout_passes=False,
                                             use_tc_tiling_on_sc=False),
        scratch_types=[
            pltpu.VMEM((LPI, BW), jnp.int32),
            pltpu.VMEM((LPI, BW), jnp.int32),
            pltpu.VMEM((ROWS, D), jnp.float32),
            pltpu.VMEM((ROWS, D), jnp.float32),
            pltpu.VMEM((TPI, 1024), jnp.float32),
            pltpu.VMEM((TPI, 1024), jnp.float32),
            pltpu.VMEM((L, D), jnp.float32),
            pltpu.VMEM((D,), jnp.float32),
            pltpu.VMEM((D,), jnp.float32),
            pltpu.SemaphoreType.DMA,
            pltpu.SemaphoreType.DMA,
            pltpu.SemaphoreType.DMA,
            pltpu.SemaphoreType.DMA,
            pltpu.SemaphoreType.DMA,
            pltpu.SemaphoreType.DMA,
        ],
    )(ids_t, gene_table, pos_table, ln_gamma, ln_beta)
    out = (out5.reshape(L, D // 8, NW, 8, BW)
           .transpose(2, 4, 0, 1, 3)
           .reshape(B, L, D))
    return out


# speedup vs baseline: 2.0062x; 2.0062x over previous
import jax
import jax.numpy as jnp
from jax import lax
from jax.experimental import pallas as pl
from jax.experimental.pallas import tpu as pltpu
from jax.experimental.pallas import tpu_sc as plsc

B = 4096
L = 200
D = 64
EPS = 1e-5

NC = 2
NS = 16
NW = NC * NS

BW = B // NW
LPI = 2
NIT = L // LPI
ROWS = LPI * BW
TPI = LPI * (D // 8)
NT = L * (D // 8)

MAGIC = 0x5F3759DF


def _rsqrt(v):
    i = plsc.bitcast(v, jnp.int32)
    i = MAGIC - lax.shift_right_logical(i, 1)
    y = plsc.bitcast(i, jnp.float32)
    hv = 0.5 * v
    y = y * (1.5 - hv * y * y)
    y = y * (1.5 - hv * y * y)
    y = y * (1.5 - hv * y * y)
    return y


def _sc_kernel(idsT_hbm, table_hbm, pos_hbm, gam_hbm, bet_hbm, out_hbm,
               idxA, idxB, inA, inB, outA, outB, ybuf, pos_v, gam_v, bet_v,
               gsemA, gsemB, osemA, osemB, isemA, isemB):
    cid = lax.axis_index("c")
    sid = lax.axis_index("s")
    wid = cid * NS + sid
    b0 = wid * BW

    idx = [idxA, idxB]
    inb = [inA, inB]
    outb = [outA, outB]
    gsem = [gsemA, gsemB]
    osem = [osemA, osemB]
    isem = [isemA, isemB]

    pltpu.sync_copy(pos_hbm.at[pl.ds(0, L), :], pos_v)
    pltpu.sync_copy(gam_hbm, gam_v)
    pltpu.sync_copy(bet_hbm, bet_v)

    g = [gam_v[pl.ds(16 * k, 16)] for k in range(4)]
    bt = [bet_v[pl.ds(16 * k, 16)] for k in range(4)]

    iota = lax.iota(jnp.int32, 16)

    def fire_ids(t, q):
        pltpu.async_copy(idsT_hbm.at[pl.ds(t * LPI, LPI), pl.ds(b0, BW)],
                         idx[q], isem[q])

    def wait_ids(q):
        pltpu.make_async_copy(idsT_hbm.at[pl.ds(0, LPI), pl.ds(0, BW)],
                              idx[q], isem[q]).wait()

    def fire_gather(q):
        for j in range(LPI):
            pltpu.async_copy(table_hbm.at[idx[q].at[j]],
                             inb[q].at[pl.ds(j * BW, BW), :], gsem[q])

    def wait_gather(q):
        pltpu.make_async_copy(table_hbm.at[pl.ds(0, ROWS), :], inb[q],
                              gsem[q]).wait()

    def fire_out(t, q):
        pltpu.async_copy(outb[q], out_hbm.at[pl.ds(t * TPI, TPI), wid],
                         osem[q])

    def wait_out(q):
        pltpu.make_async_copy(out_hbm.at[pl.ds(0, TPI), 0], outb[q],
                              osem[q]).wait()

    fire_ids(0, 0)
    wait_ids(0)
    fire_ids(1, 1)
    fire_gather(0)

    @pl.loop(0, NIT, step=2)
    def _iter2(t0):
        for p in (0, 1):
            t = t0 + p
            q = 1 - p

            @pl.when(t < NIT - 1)
            def _prefetch():
                wait_ids(q)
                fire_gather(q)

            wait_gather(p)

            @pl.when(t < NIT - 2)
            def _nextids():
                fire_ids(t + 2, p)

            @pl.when(t >= 2)
            def _drainout():
                wait_out(p)

            src = inb[p]
            dst = outb[p]

            for j in range(LPI):
                l = t * LPI + j
                pv = [pos_v[l, pl.ds(16 * k, 16)] for k in range(4)]

                @plsc.parallel_loop(0, BW, unroll=4)
                def _row(r):
                    row = j * BW + r
                    x = [src[row, pl.ds(16 * k, 16)] + pv[k]
                         for k in range(4)]
                    tot = (x[0] + x[1]) + (x[2] + x[3])
                    qq = ((x[0] * x[0] + x[1] * x[1])
                          + (x[2] * x[2] + x[3] * x[3]))
                    sv = jnp.full((16,), jnp.sum(tot))
                    qv = jnp.full((16,), jnp.sum(qq))
                    mean = sv * (1.0 / D)
                    var = qv * (1.0 / D) - mean * mean
                    rstd = _rsqrt(var + EPS)
                    for k in range(4):
                        y = (x[k] - mean) * (rstd * g[k]) + bt[k]
                        ybuf[row, pl.ds(16 * k, 16)] = y

            for j in range(LPI):
                rbase = j * BW

                @plsc.parallel_loop(0, 8 * D, unroll=8)
                def _tp(qi):
                    dd = lax.shift_right_logical(qi, 3)
                    bg = jnp.bitwise_and(qi, 7)
                    rows = jnp.full((16,), rbase + bg * 16, jnp.int32) + iota
                    cols = jnp.full((16,), dd, jnp.int32)
                    y = plsc.load_gather(ybuf, [rows, cols])
                    tl = j * 8 + lax.shift_right_logical(dd, 3)
                    co = jnp.bitwise_and(dd, 7) * BW + bg * 16
                    dst[tl, pl.ds(co, 16)] = y

            fire_out(t, p)

    wait_out(0)
    wait_out(1)


@jax.jit
def kernel(input_ids_BL, gene_table, pos_table, ln_gamma, ln_beta):
    ids_t = input_ids_BL.astype(jnp.int32).T

    mesh = plsc.VectorSubcoreMesh(core_axis_name="c", subcore_axis_name="s",
                                  num_cores=NC, num_subcores=NS)
    out5 = pl.kernel(
        _sc_kernel,
        out_type=jax.ShapeDtypeStruct((NT, NW, 1024), jnp.float32),
        mesh=mesh,
        compiler_params=pltpu.CompilerParams(needs_layout_passes=False,
                                             use_tc_tiling_on_sc=False),
        scratch_types=[
            pltpu.VMEM((LPI, BW), jnp.int32),
            pltpu.VMEM((LPI, BW), jnp.int32),
            pltpu.VMEM((ROWS, D), jnp.float32),
            pltpu.VMEM((ROWS, D), jnp.float32),
            pltpu.VMEM((TPI, 1024), jnp.float32),
            pltpu.VMEM((TPI, 1024), jnp.float32),
            pltpu.VMEM((ROWS, 65), jnp.float32),
            pltpu.VMEM((L, D), jnp.float32),
            pltpu.VMEM((D,), jnp.float32),
            pltpu.VMEM((D,), jnp.float32),
            pltpu.SemaphoreType.DMA,
            pltpu.SemaphoreType.DMA,
            pltpu.SemaphoreType.DMA,
            pltpu.SemaphoreType.DMA,
            pltpu.SemaphoreType.DMA,
            pltpu.SemaphoreType.DMA,
        ],
    )(ids_t, gene_table, pos_table, ln_gamma, ln_beta)
    out = (out5.reshape(L, D // 8, NW, 8, BW)
           .transpose(2, 4, 0, 1, 3)
           .reshape(B, L, D))
    return out
